# baseline (speedup 1.0000x reference)
import functools

import jax
import jax.numpy as jnp
from jax import lax
from jax.experimental import pallas as pl
from jax.experimental.pallas import tpu as pltpu
from jax.experimental.pallas import tpu_sc as plsc

NFEATS = 26
NMEM = 200
KOUT = 8
BATCH = 4096

NW = 32
ROWS_PER_W = BATCH // NW
RCHUNK = 8
NCHUNKS = ROWS_PER_W // RCHUNK
ROWLEN = NFEATS * NMEM
OUTLEN = NFEATS * KOUT
JUNROLL = 8


def _topk_body(x_hbm, a_hbm, b_hbm, out_hbm,
               av, bv, xb0, xb1, ob, sem0, sem1, sem_out):
    nc = 2
    wid = lax.axis_index("s") * nc + lax.axis_index("c")
    row0 = wid * ROWS_PER_W

    pltpu.sync_copy(a_hbm, av)
    pltpu.sync_copy(b_hbm, bv)

    lane = lax.iota(jnp.int32, 16)
    lane8 = lane * KOUT
    neginf = jnp.full((16,), -jnp.inf, jnp.float32)

    def in_copy(c, buf, sem):
        rowbase = row0 + c * RCHUNK
        return pltpu.make_async_copy(
            x_hbm.at[pl.ds(rowbase, RCHUNK)], buf, sem)

    def compute_slab(xb, c):
        rowbase = row0 + c * RCHUNK

        def row_body(r, carry):
            rowvec = jnp.full((16,), 0, jnp.int32) + r
            for f0 in (0, 13):
                af = av[pl.ds(f0, 16)]
                bf = bv[pl.ds(f0, 16)]
                sflip = jnp.where(af < 0, -1.0, 1.0).astype(jnp.float32)
                absa = af * sflip

                def elem(colv, regs):
                    g = plsc.load_gather(xb, [rowvec, colv])
                    z = sflip * g
                    out = []
                    for d in range(KOUT):
                        rr = regs[d]
                        if d < KOUT - 1:
                            out.append(jnp.maximum(rr, z))
                            z = jnp.minimum(rr, z)
                        else:
                            out.append(jnp.maximum(rr, z))
                    nxt = colv + NFEATS
                    if f0 != 0:
                        nxt = jnp.minimum(nxt, ROWLEN - 1)
                    return nxt, tuple(out)

                def jo_body(jo, carry_in):
                    colv, regs = carry_in
                    for _ in range(JUNROLL):
                        colv, regs = elem(colv, regs)
                    return (colv, regs)

                _, regs = lax.fori_loop(
                    0, NMEM // JUNROLL, jo_body,
                    (lane + f0, (neginf,) * KOUT))

                mask = None if f0 == 0 else (lane < (NFEATS - 13))
                for d in range(KOUT):
                    v = absa * regs[d] + bf
                    plsc.store_scatter(ob, [rowvec, lane8 + (f0 * KOUT + d)],
                                       v, mask=mask)
            return carry

        lax.fori_loop(0, RCHUNK, row_body, 0)
        pltpu.async_copy(ob, out_hbm.at[pl.ds(rowbase, RCHUNK)],
                         sem_out).wait()

    in_copy(0, xb0, sem0).start()

    def pair_body(g, carry):
        c0 = 2 * g
        in_copy(c0 + 1, xb1, sem1).start()
        in_copy(c0, xb0, sem0).wait()
        compute_slab(xb0, c0)

        @pl.when(g < NCHUNKS // 2 - 1)
        def _():
            in_copy(c0 + 2, xb0, sem0).start()

        in_copy(c0 + 1, xb1, sem1).wait()
        compute_slab(xb1, c0 + 1)
        return carry

    lax.fori_loop(0, NCHUNKS // 2, pair_body, 0)


@jax.jit
def _sc_topk(x, a32, b32):
    mesh = plsc.VectorSubcoreMesh(core_axis_name="c", subcore_axis_name="s")
    f = functools.partial(
        pl.kernel,
        out_type=jax.ShapeDtypeStruct((BATCH, OUTLEN), jnp.float32),
        mesh=mesh,
        scratch_types=[
            pltpu.VMEM((32,), jnp.float32),
            pltpu.VMEM((32,), jnp.float32),
            pltpu.VMEM((RCHUNK, ROWLEN), jnp.float32),
            pltpu.VMEM((RCHUNK, ROWLEN), jnp.float32),
            pltpu.VMEM((RCHUNK, OUTLEN), jnp.float32),
            pltpu.SemaphoreType.DMA,
            pltpu.SemaphoreType.DMA,
            pltpu.SemaphoreType.DMA,
        ],
        compiler_params=pltpu.CompilerParams(needs_layout_passes=False),
    )(_topk_body)
    return f(x, a32, b32)


def kernel(x, alpha, beta):
    a32 = jnp.pad(alpha, (0, 32 - NFEATS))
    b32 = jnp.pad(beta, (0, 32 - NFEATS))
    return _sc_topk(x, a32, b32)


# speedup vs baseline: 1.2377x; 1.2377x over previous
import functools

import jax
import jax.numpy as jnp
from jax import lax
from jax.experimental import pallas as pl
from jax.experimental.pallas import tpu as pltpu
from jax.experimental.pallas import tpu_sc as plsc

NFEATS = 26
NMEM = 200
KOUT = 8
BATCH = 4096

NW = 32
ROWS_PER_W = BATCH // NW
RCHUNK = 8
NCHUNKS = ROWS_PER_W // RCHUNK
ROWLEN = NFEATS * NMEM
OUTLEN = NFEATS * KOUT
JUNROLL = 8

SORT8 = ((0, 1), (2, 3), (4, 5), (6, 7),
         (0, 2), (1, 3), (4, 6), (5, 7),
         (1, 2), (5, 6),
         (0, 4), (1, 5), (2, 6), (3, 7),
         (2, 4), (3, 5),
         (1, 2), (3, 4), (5, 6))
BITONIC8 = ((0, 4), (1, 5), (2, 6), (3, 7),
            (0, 2), (1, 3), (4, 6), (5, 7),
            (0, 1), (2, 3), (4, 5), (6, 7))


def _topk_body(x_hbm, a_hbm, b_hbm, out_hbm,
               av, bv, xb0, xb1, ob, sem0, sem1, sem_out):
    nc = 2
    wid = lax.axis_index("s") * nc + lax.axis_index("c")
    row0 = wid * ROWS_PER_W

    pltpu.sync_copy(a_hbm, av)
    pltpu.sync_copy(b_hbm, bv)

    lane = lax.iota(jnp.int32, 16)
    lane8 = lane * KOUT
    neginf = jnp.full((16,), -jnp.inf, jnp.float32)

    def in_copy(c, buf, sem):
        rowbase = row0 + c * RCHUNK
        return pltpu.make_async_copy(
            x_hbm.at[pl.ds(rowbase, RCHUNK)], buf, sem)

    def compute_slab(xb, c):
        rowbase = row0 + c * RCHUNK

        def row_body(r, carry):
            rowvec = jnp.full((16,), 0, jnp.int32) + r
            for f0 in (0, 13):
                af = av[pl.ds(f0, 16)]
                bf = bv[pl.ds(f0, 16)]
                sflip = jnp.where(af < 0, -1.0, 1.0).astype(jnp.float32)
                absa = af * sflip

                def jo_body(jo, carry_in):
                    colv, regs = carry_in
                    f = []
                    for k in range(JUNROLL):
                        ck = colv + k * NFEATS
                        if f0 != 0:
                            ck = jnp.minimum(ck, ROWLEN - 1)
                        f.append(sflip * plsc.load_gather(xb, [rowvec, ck]))
                    for (i, j) in SORT8:
                        lo = jnp.minimum(f[i], f[j])
                        f[j] = jnp.maximum(f[i], f[j])
                        f[i] = lo
                    h = [jnp.maximum(regs[d], f[d]) for d in range(KOUT)]
                    for (i, j) in BITONIC8:
                        hi = jnp.maximum(h[i], h[j])
                        h[j] = jnp.minimum(h[i], h[j])
                        h[i] = hi
                    return (colv + JUNROLL * NFEATS, tuple(h))

                _, regs = lax.fori_loop(
                    0, NMEM // JUNROLL, jo_body,
                    (lane + f0, (neginf,) * KOUT))

                mask = None if f0 == 0 else (lane < (NFEATS - 13))
                for d in range(KOUT):
                    v = absa * regs[d] + bf
                    plsc.store_scatter(ob, [rowvec, lane8 + (f0 * KOUT + d)],
                                       v, mask=mask)
            return carry

        lax.fori_loop(0, RCHUNK, row_body, 0)
        pltpu.async_copy(ob, out_hbm.at[pl.ds(rowbase, RCHUNK)],
                         sem_out).wait()

    in_copy(0, xb0, sem0).start()

    def pair_body(g, carry):
        c0 = 2 * g
        in_copy(c0 + 1, xb1, sem1).start()
        in_copy(c0, xb0, sem0).wait()
        compute_slab(xb0, c0)

        @pl.when(g < NCHUNKS // 2 - 1)
        def _():
            in_copy(c0 + 2, xb0, sem0).start()

        in_copy(c0 + 1, xb1, sem1).wait()
        compute_slab(xb1, c0 + 1)
        return carry

    lax.fori_loop(0, NCHUNKS // 2, pair_body, 0)


@jax.jit
def _sc_topk(x, a32, b32):
    mesh = plsc.VectorSubcoreMesh(core_axis_name="c", subcore_axis_name="s")
    f = functools.partial(
        pl.kernel,
        out_type=jax.ShapeDtypeStruct((BATCH, OUTLEN), jnp.float32),
        mesh=mesh,
        scratch_types=[
            pltpu.VMEM((32,), jnp.float32),
            pltpu.VMEM((32,), jnp.float32),
            pltpu.VMEM((RCHUNK, ROWLEN), jnp.float32),
            pltpu.VMEM((RCHUNK, ROWLEN), jnp.float32),
            pltpu.VMEM((RCHUNK, OUTLEN), jnp.float32),
            pltpu.SemaphoreType.DMA,
            pltpu.SemaphoreType.DMA,
            pltpu.SemaphoreType.DMA,
        ],
        compiler_params=pltpu.CompilerParams(needs_layout_passes=False),
    )(_topk_body)
    return f(x, a32, b32)


def kernel(x, alpha, beta):
    a32 = jnp.pad(alpha, (0, 32 - NFEATS))
    b32 = jnp.pad(beta, (0, 32 - NFEATS))
    return _sc_topk(x, a32, b32)
